# baseline (speedup 1.0000x reference)
import functools

import jax
import jax.numpy as jnp
from jax import lax
from jax.experimental import pallas as pl
from jax.experimental.pallas import tpu as pltpu
from jax.experimental.pallas import tpu_sc as plsc

B = 16384
VOCAB = 100000
EMB = 64
NG = 19
NGR = 30
NC_CAT = 20
NCAT = NG + NGR + NC_CAT
CATP = 128
FD = 128
EPS = 1e-3

IDXW = 128
BLK = 512
NBLK = B // BLK


def _make_gather():
    info = plsc.get_sparse_core_info()
    num_cores, num_subcores = info.num_cores, info.num_subcores
    nw = num_cores * num_subcores
    bpw = B // nw
    chunks = bpw // IDXW
    rows2d = B // IDXW

    mesh = plsc.VectorSubcoreMesh(core_axis_name="c", subcore_axis_name="s")

    @functools.partial(
        pl.kernel,
        out_type=(
            jax.ShapeDtypeStruct((B, EMB), jnp.float32),
            jax.ShapeDtypeStruct((rows2d, IDXW), jnp.int32),
            jax.ShapeDtypeStruct((rows2d, IDXW), jnp.int32),
            jax.ShapeDtypeStruct((rows2d, IDXW), jnp.int32),
        ),
        mesh=mesh,
        compiler_params=pltpu.CompilerParams(use_tc_tiling_on_sc=False),
        scratch_types=[
            pltpu.VMEM((chunks, IDXW), jnp.int32),
            pltpu.VMEM((bpw, EMB), jnp.float32),
            pltpu.VMEM((chunks, IDXW), jnp.int32),
            pltpu.VMEM((chunks, IDXW), jnp.int32),
            pltpu.VMEM((chunks, IDXW), jnp.int32),
            pltpu.SemaphoreType.DMA,
        ],
    )
    def gather(ids2d, emb_hbm, gmap, grmap, cmap,
               emb_out, g_out, gr_out, c_out,
               idx_v, rows_v, g_v, gr_v, c_v, sem):
        wid = lax.axis_index("s") * num_cores + lax.axis_index("c")
        base = wid * chunks

        pltpu.sync_copy(ids2d.at[pl.ds(base, chunks)], idx_v)

        cps = []
        for j in range(chunks):
            idx_j = idx_v.at[j]
            cps.append(pltpu.async_copy(
                emb_hbm.at[idx_j], rows_v.at[pl.ds(j * IDXW, IDXW)], sem))
            cps.append(pltpu.async_copy(gmap.at[idx_j], g_v.at[j], sem))
            cps.append(pltpu.async_copy(grmap.at[idx_j], gr_v.at[j], sem))
            cps.append(pltpu.async_copy(cmap.at[idx_j], c_v.at[j], sem))
        for cp in cps:
            cp.wait()

        pltpu.sync_copy(rows_v, emb_out.at[pl.ds(wid * bpw, bpw)])
        pltpu.sync_copy(g_v, g_out.at[pl.ds(base, chunks)])
        pltpu.sync_copy(gr_v, gr_out.at[pl.ds(base, chunks)])
        pltpu.sync_copy(c_v, c_out.at[pl.ds(base, chunks)])

    return gather


def _tc_body(g_ref, gr_ref, c_ref, emb_ref, we_ref, wc_ref,
             ge_ref, be_ref, me_ref, ve_ref,
             gc_ref, bc_ref, mc_ref, vc_ref, out_ref):
    g = g_ref[0]
    gr = gr_ref[0]
    c = c_ref[0]

    sub = lax.broadcasted_iota(jnp.int32, (CATP, BLK), 0)
    hot = (sub == g) | (sub == gr + NG) | (sub == c + (NG + NGR))

    s_cat = gc_ref[...] * lax.rsqrt(vc_ref[...] + EPS)
    t_cat = bc_ref[...] - mc_ref[...] * s_cat
    xcat_t = jnp.where(hot, s_cat + t_cat, t_cat)

    s_emb = ge_ref[...] * lax.rsqrt(ve_ref[...] + EPS)
    t_emb = be_ref[...] - me_ref[...] * s_emb
    xemb = emb_ref[...] * s_emb + t_emb

    acc = lax.dot_general(xemb, we_ref[...], (((1,), (0,)), ((), ())),
                          preferred_element_type=jnp.float32)
    acc = acc + lax.dot_general(xcat_t, wc_ref[...], (((0,), (0,)), ((), ())),
                                preferred_element_type=jnp.float32)
    out_ref[...] = acc


def _const2(i):
    return (0, 0)


def _const3(i):
    return (0, 0, 0)


_tc_call = pl.pallas_call(
    _tc_body,
    grid=(NBLK,),
    in_specs=[
        pl.BlockSpec((1, 1, BLK), lambda i: (i, 0, 0)),
        pl.BlockSpec((1, 1, BLK), lambda i: (i, 0, 0)),
        pl.BlockSpec((1, 1, BLK), lambda i: (i, 0, 0)),
        pl.BlockSpec((BLK, EMB), lambda i: (i, 0)),
        pl.BlockSpec((EMB, FD), _const2),
        pl.BlockSpec((CATP, FD), _const2),
        pl.BlockSpec((1, EMB), _const2),
        pl.BlockSpec((1, EMB), _const2),
        pl.BlockSpec((1, EMB), _const2),
        pl.BlockSpec((1, EMB), _const2),
        pl.BlockSpec((CATP, 1), _const2),
        pl.BlockSpec((CATP, 1), _const2),
        pl.BlockSpec((CATP, 1), _const2),
        pl.BlockSpec((CATP, 1), _const2),
    ],
    out_specs=pl.BlockSpec((BLK, FD), lambda i: (i, 0)),
    out_shape=jax.ShapeDtypeStruct((B, FD), jnp.float32),
)


def kernel(article_id, emb_table, group_map, graphical_map, colour_map,
           gamma, beta, moving_mean, moving_var, W):
    ids2d = article_id.reshape(B // IDXW, IDXW)
    emb_rows, g2, gr2, c2 = _make_gather()(
        ids2d, emb_table, group_map, graphical_map, colour_map)

    g3 = g2.reshape(NBLK, 1, BLK)
    gr3 = gr2.reshape(NBLK, 1, BLK)
    c3 = c2.reshape(NBLK, 1, BLK)

    pad = CATP - NCAT
    we = W[:EMB]
    wc = jnp.pad(W[EMB:], ((0, pad), (0, 0)))
    ge = gamma[:EMB].reshape(1, EMB)
    be = beta[:EMB].reshape(1, EMB)
    me = moving_mean[:EMB].reshape(1, EMB)
    ve = moving_var[:EMB].reshape(1, EMB)
    gc = jnp.pad(gamma[EMB:], (0, pad)).reshape(CATP, 1)
    bc = jnp.pad(beta[EMB:], (0, pad)).reshape(CATP, 1)
    mc = jnp.pad(moving_mean[EMB:], (0, pad)).reshape(CATP, 1)
    vc = jnp.pad(moving_var[EMB:], (0, pad), constant_values=1.0).reshape(CATP, 1)

    return _tc_call(g3, gr3, c3, emb_rows, we, wc,
                    ge, be, me, ve, gc, bc, mc, vc)


# speedup vs baseline: 1.2795x; 1.2795x over previous
import functools

import jax
import jax.numpy as jnp
from jax import lax
from jax.experimental import pallas as pl
from jax.experimental.pallas import tpu as pltpu
from jax.experimental.pallas import tpu_sc as plsc

B = 16384
VOCAB = 100000
EMB = 64
NG = 19
NGR = 30
NC_CAT = 20
NCAT = NG + NGR + NC_CAT
CATP = 128
FD = 128
EPS = 1e-3

IDXW = 128
GRP = 16
BLK = 512
NBLK = B // BLK


def _make_gather():
    info = plsc.get_sparse_core_info()
    num_cores, num_subcores = info.num_cores, info.num_subcores
    nw = num_cores * num_subcores
    bpw = B // nw
    chunks = bpw // IDXW
    ngrp = bpw // GRP

    mesh = plsc.VectorSubcoreMesh(core_axis_name="c", subcore_axis_name="s")

    @functools.partial(
        pl.kernel,
        out_type=(
            jax.ShapeDtypeStruct((B, EMB), jnp.float32),
            jax.ShapeDtypeStruct((nw, 8, bpw), jnp.int32),
        ),
        mesh=mesh,
        scratch_types=[
            pltpu.VMEM((bpw,), jnp.int32),
            pltpu.VMEM((bpw, EMB), jnp.float32),
            pltpu.VMEM((8, bpw), jnp.int32),
            pltpu.SemaphoreType.DMA,
            pltpu.SemaphoreType.DMA,
        ],
    )
    def gather(ids_hbm, emb_hbm, gmap, grmap, cmap,
               emb_out, cats_out,
               idx_v, rows_v, cats_v, sem, sem2):
        wid = lax.axis_index("s") * num_cores + lax.axis_index("c")
        base = wid * bpw

        pltpu.sync_copy(ids_hbm.at[pl.ds(base, bpw)], idx_v)

        map_cps = []
        for c in range(chunks):
            sl = pl.ds(c * IDXW, IDXW)
            map_cps.append(pltpu.async_copy(
                gmap.at[idx_v.at[sl]], cats_v.at[0, sl], sem))
            map_cps.append(pltpu.async_copy(
                grmap.at[idx_v.at[sl]], cats_v.at[1, sl], sem))
            map_cps.append(pltpu.async_copy(
                cmap.at[idx_v.at[sl]], cats_v.at[2, sl], sem))

        def body(g, carry):
            vec = idx_v[pl.ds(g * GRP, GRP)]
            grp_cps = []
            for jj in range(GRP):
                v = vec[jj]
                grp_cps.append(pltpu.async_copy(
                    emb_hbm.at[pl.ds(v, 1)],
                    rows_v.at[pl.ds(g * GRP + jj, 1)], sem2))

            @pl.when(g > 0)
            def _():
                for cp in grp_cps:
                    cp.wait()

            return carry

        lax.fori_loop(0, ngrp, body, 0)

        for jj in range(GRP):
            pltpu.make_async_copy(
                emb_hbm.at[pl.ds(0, 1)],
                rows_v.at[pl.ds(jj, 1)], sem2).wait()
        for cp in map_cps:
            cp.wait()

        pltpu.sync_copy(rows_v, emb_out.at[pl.ds(base, bpw)])
        pltpu.sync_copy(cats_v, cats_out.at[wid])

    return gather


def _tc_body(cats_ref, emb_ref, we_ref, wc_ref,
             ge_ref, be_ref, me_ref, ve_ref,
             gc_ref, bc_ref, mc_ref, vc_ref, out_ref):
    cats = cats_ref[0]
    g = cats[0:1, :]
    gr = cats[1:2, :]
    c = cats[2:3, :]

    sub = lax.broadcasted_iota(jnp.int32, (CATP, BLK), 0)
    hot = (sub == g) | (sub == gr + NG) | (sub == c + (NG + NGR))

    s_cat = gc_ref[...] * lax.rsqrt(vc_ref[...] + EPS)
    t_cat = bc_ref[...] - mc_ref[...] * s_cat
    xcat_t = jnp.where(hot, s_cat + t_cat, t_cat)

    s_emb = ge_ref[...] * lax.rsqrt(ve_ref[...] + EPS)
    t_emb = be_ref[...] - me_ref[...] * s_emb
    xemb = emb_ref[...] * s_emb + t_emb

    acc = lax.dot_general(xemb, we_ref[...], (((1,), (0,)), ((), ())),
                          preferred_element_type=jnp.float32)
    acc = acc + lax.dot_general(xcat_t, wc_ref[...], (((0,), (0,)), ((), ())),
                                preferred_element_type=jnp.float32)
    out_ref[...] = acc


def _const2(i):
    return (0, 0)


_tc_call = pl.pallas_call(
    _tc_body,
    grid=(NBLK,),
    in_specs=[
        pl.BlockSpec((1, 8, BLK), lambda i: (i, 0, 0)),
        pl.BlockSpec((BLK, EMB), lambda i: (i, 0)),
        pl.BlockSpec((EMB, FD), _const2),
        pl.BlockSpec((CATP, FD), _const2),
        pl.BlockSpec((1, EMB), _const2),
        pl.BlockSpec((1, EMB), _const2),
        pl.BlockSpec((1, EMB), _const2),
        pl.BlockSpec((1, EMB), _const2),
        pl.BlockSpec((CATP, 1), _const2),
        pl.BlockSpec((CATP, 1), _const2),
        pl.BlockSpec((CATP, 1), _const2),
        pl.BlockSpec((CATP, 1), _const2),
    ],
    out_specs=pl.BlockSpec((BLK, FD), lambda i: (i, 0)),
    out_shape=jax.ShapeDtypeStruct((B, FD), jnp.float32),
)


def kernel(article_id, emb_table, group_map, graphical_map, colour_map,
           gamma, beta, moving_mean, moving_var, W):
    emb_rows, cats = _make_gather()(
        article_id, emb_table, group_map, graphical_map, colour_map)

    pad = CATP - NCAT
    we = W[:EMB]
    wc = jnp.pad(W[EMB:], ((0, pad), (0, 0)))
    ge = gamma[:EMB].reshape(1, EMB)
    be = beta[:EMB].reshape(1, EMB)
    me = moving_mean[:EMB].reshape(1, EMB)
    ve = moving_var[:EMB].reshape(1, EMB)
    gc = jnp.pad(gamma[EMB:], (0, pad)).reshape(CATP, 1)
    bc = jnp.pad(beta[EMB:], (0, pad)).reshape(CATP, 1)
    mc = jnp.pad(moving_mean[EMB:], (0, pad)).reshape(CATP, 1)
    vc = jnp.pad(moving_var[EMB:], (0, pad), constant_values=1.0).reshape(CATP, 1)

    return _tc_call(cats, emb_rows, we, wc, ge, be, me, ve, gc, bc, mc, vc)


# speedup vs baseline: 1.4919x; 1.1661x over previous
import functools

import jax
import jax.numpy as jnp
from jax import lax
from jax.experimental import pallas as pl
from jax.experimental.pallas import tpu as pltpu
from jax.experimental.pallas import tpu_sc as plsc

B = 16384
VOCAB = 100000
EMB = 64
NG = 19
NGR = 30
NC_CAT = 20
NCAT = NG + NGR + NC_CAT
CATP = 128
FD = 128
EPS = 1e-3

IDXW = 128
GRP = 16
BLK = 2048
SUBB = 512
NSUB = BLK // SUBB
NBLK = B // BLK


def _make_gather():
    info = plsc.get_sparse_core_info()
    num_cores, num_subcores = info.num_cores, info.num_subcores
    nw = num_cores * num_subcores
    bpw = B // nw
    chunks = bpw // IDXW
    ngrp = bpw // GRP

    mesh = plsc.VectorSubcoreMesh(core_axis_name="c", subcore_axis_name="s")

    @functools.partial(
        pl.kernel,
        out_type=(
            jax.ShapeDtypeStruct((B, EMB), jnp.float32),
            jax.ShapeDtypeStruct((nw, 8, bpw), jnp.int32),
        ),
        mesh=mesh,
        scratch_types=[
            pltpu.VMEM((bpw,), jnp.int32),
            pltpu.VMEM((bpw, EMB), jnp.float32),
            pltpu.VMEM((8, bpw), jnp.int32),
            pltpu.SemaphoreType.DMA,
            pltpu.SemaphoreType.DMA,
        ],
    )
    def gather(ids_hbm, emb_hbm, gmap, grmap, cmap,
               emb_out, cats_out,
               idx_v, rows_v, cats_v, sem, sem2):
        wid = lax.axis_index("s") * num_cores + lax.axis_index("c")
        base = wid * bpw

        pltpu.sync_copy(ids_hbm.at[pl.ds(base, bpw)], idx_v)

        map_cps = []
        for c in range(chunks):
            sl = pl.ds(c * IDXW, IDXW)
            map_cps.append(pltpu.async_copy(
                gmap.at[idx_v.at[sl]], cats_v.at[0, sl], sem))
            map_cps.append(pltpu.async_copy(
                grmap.at[idx_v.at[sl]], cats_v.at[1, sl], sem))
            map_cps.append(pltpu.async_copy(
                cmap.at[idx_v.at[sl]], cats_v.at[2, sl], sem))

        def body(g, carry):
            vec = idx_v[pl.ds(g * GRP, GRP)]
            grp_cps = []
            for jj in range(GRP):
                v = vec[jj]
                grp_cps.append(pltpu.async_copy(
                    emb_hbm.at[pl.ds(v, 1)],
                    rows_v.at[pl.ds(g * GRP + jj, 1)], sem2))

            @pl.when(g > 0)
            def _():
                for cp in grp_cps:
                    cp.wait()

            return carry

        lax.fori_loop(0, ngrp, body, 0)

        for jj in range(GRP):
            pltpu.make_async_copy(
                emb_hbm.at[pl.ds(0, 1)],
                rows_v.at[pl.ds(jj, 1)], sem2).wait()
        for cp in map_cps:
            cp.wait()

        pltpu.sync_copy(rows_v, emb_out.at[pl.ds(base, bpw)])
        pltpu.sync_copy(cats_v, cats_out.at[wid])

    return gather


def _tc_body(cats_ref, emb_ref, we_ref, wc_ref,
             ge_ref, be_ref, me_ref, ve_ref,
             gc_ref, bc_ref, mc_ref, vc_ref, out_ref):
    g = jnp.concatenate([cats_ref[k, 0:1, :] for k in range(NSUB)], axis=1)
    gr = jnp.concatenate([cats_ref[k, 1:2, :] for k in range(NSUB)], axis=1)
    c = jnp.concatenate([cats_ref[k, 2:3, :] for k in range(NSUB)], axis=1)

    sub = lax.broadcasted_iota(jnp.int32, (CATP, BLK), 0)
    hot = (sub == g) | (sub == gr + NG) | (sub == c + (NG + NGR))

    s_cat = gc_ref[...] * lax.rsqrt(vc_ref[...] + EPS)
    t_cat = bc_ref[...] - mc_ref[...] * s_cat
    xcat_t = jnp.where(hot, s_cat + t_cat, t_cat)

    s_emb = ge_ref[...] * lax.rsqrt(ve_ref[...] + EPS)
    t_emb = be_ref[...] - me_ref[...] * s_emb
    xemb = emb_ref[...] * s_emb + t_emb

    acc = lax.dot_general(xemb, we_ref[...], (((1,), (0,)), ((), ())),
                          preferred_element_type=jnp.float32)
    acc = acc + lax.dot_general(xcat_t, wc_ref[...], (((0,), (0,)), ((), ())),
                                preferred_element_type=jnp.float32)
    out_ref[...] = acc


def _const2(i):
    return (0, 0)


_tc_call = pl.pallas_call(
    _tc_body,
    grid=(NBLK,),
    in_specs=[
        pl.BlockSpec((NSUB, 8, SUBB), lambda i: (i, 0, 0)),
        pl.BlockSpec((BLK, EMB), lambda i: (i, 0)),
        pl.BlockSpec((EMB, FD), _const2),
        pl.BlockSpec((CATP, FD), _const2),
        pl.BlockSpec((1, EMB), _const2),
        pl.BlockSpec((1, EMB), _const2),
        pl.BlockSpec((1, EMB), _const2),
        pl.BlockSpec((1, EMB), _const2),
        pl.BlockSpec((CATP, 1), _const2),
        pl.BlockSpec((CATP, 1), _const2),
        pl.BlockSpec((CATP, 1), _const2),
        pl.BlockSpec((CATP, 1), _const2),
    ],
    out_specs=pl.BlockSpec((BLK, FD), lambda i: (i, 0)),
    out_shape=jax.ShapeDtypeStruct((B, FD), jnp.float32),
)


def kernel(article_id, emb_table, group_map, graphical_map, colour_map,
           gamma, beta, moving_mean, moving_var, W):
    emb_rows, cats = _make_gather()(
        article_id, emb_table, group_map, graphical_map, colour_map)

    pad = CATP - NCAT
    we = W[:EMB]
    wc = jnp.pad(W[EMB:], ((0, pad), (0, 0)))
    ge = gamma[:EMB].reshape(1, EMB)
    be = beta[:EMB].reshape(1, EMB)
    me = moving_mean[:EMB].reshape(1, EMB)
    ve = moving_var[:EMB].reshape(1, EMB)
    gc = jnp.pad(gamma[EMB:], (0, pad)).reshape(CATP, 1)
    bc = jnp.pad(beta[EMB:], (0, pad)).reshape(CATP, 1)
    mc = jnp.pad(moving_mean[EMB:], (0, pad)).reshape(CATP, 1)
    vc = jnp.pad(moving_var[EMB:], (0, pad), constant_values=1.0).reshape(CATP, 1)

    return _tc_call(cats, emb_rows, we, wc, ge, be, me, ve, gc, bc, mc, vc)
